# Initial kernel scaffold; baseline (speedup 1.0000x reference)
import jax
import jax.numpy as jnp
from jax.experimental import pallas as pl


def kernel(x, edge_index, Wl0, bl0, Wr0, gamma0, beta0, Wl1, bl1, Wr1):
    raise NotImplementedError("write your pallas kernel here")


# speedup vs baseline: 4.7600x; 4.7600x over previous
import functools

import jax
import jax.numpy as jnp
from jax import lax
from jax.experimental import pallas as pl
from jax.experimental.pallas import tpu as pltpu
from jax.experimental.pallas import tpu_sc as plsc

N_NODES = 10000
N_EDGES = 320000
D_IN = 128
D_HID = 128
D_OUT = 64
EPS = 1e-5

_NC = 2
_NS = 16
_NW = _NC * _NS
_EPW = N_EDGES // _NW
_CH = 80
_NCHUNK = _EPW // _CH
_RPT = 624
_TAIL0 = _NS * _RPT
_ZROWS = 16
_D = 128


def _fill_f32(ref, rows, width, val):
    vec = jnp.full((16,), val, jnp.float32)

    def body(i, _):
        for k in range(width // 16):
            ref[i, pl.ds(k * 16, 16)] = vec
        return 0

    lax.fori_loop(0, rows, body, 0)


def _mesh():
    return plsc.VectorSubcoreMesh(core_axis_name="c", subcore_axis_name="s",
                                  num_cores=_NC, num_subcores=_NS)


def _zero_acc(sid, r0, zbuf, acc):
    _fill_f32(zbuf, _ZROWS, _D, 0.0)

    def zloop(j, _):
        pltpu.sync_copy(zbuf, acc.at[pl.ds(r0 + j * _ZROWS, _ZROWS)])
        return 0

    lax.fori_loop(0, _RPT // _ZROWS, zloop, 0)

    @pl.when(sid == _NS - 1)
    def _():
        pltpu.sync_copy(zbuf, acc.at[pl.ds(_TAIL0, _ZROWS)])


def _publish(sid, cid, r0, acc, out):
    pltpu.sync_copy(acc.at[pl.ds(r0, _RPT)], out.at[cid, pl.ds(r0, _RPT)])

    @pl.when(sid == _NS - 1)
    def _():
        pltpu.sync_copy(acc.at[pl.ds(_TAIL0, _ZROWS)],
                        out.at[cid, pl.ds(_TAIL0, _ZROWS)])


@functools.lru_cache(maxsize=None)
def _make_seg_sum():
    out_type = [jax.ShapeDtypeStruct((_NC, N_NODES, _D), jnp.float32)]
    scratch = [
        pltpu.VMEM_SHARED((N_NODES, _D), jnp.float32),
        pltpu.VMEM((_ZROWS, _D), jnp.float32),
        pltpu.VMEM((_CH,), jnp.int32),
        pltpu.VMEM((_CH,), jnp.int32),
        pltpu.VMEM((_CH, _D), jnp.float32),
        pltpu.SemaphoreType.DMA,
    ]

    def body(x_hbm, src_hbm, dst_hbm, sum_out, acc, zbuf, sidx, didx, rows, sem):
        cid = lax.axis_index("c")
        sid = lax.axis_index("s")
        wid = sid * _NC + cid
        r0 = sid * _RPT
        _zero_acc(sid, r0, zbuf, acc)
        plsc.subcore_barrier()

        base = wid * _EPW

        def eloop(i, _):
            off = base + i * _CH
            pltpu.sync_copy(src_hbm.at[pl.ds(off, _CH)], sidx)
            pltpu.sync_copy(dst_hbm.at[pl.ds(off, _CH)], didx)
            pltpu.async_copy(x_hbm.at[sidx], rows, sem).wait()
            pltpu.sync_copy(rows, acc.at[didx], add=True)
            return 0

        lax.fori_loop(0, _NCHUNK, eloop, 0)
        plsc.subcore_barrier()
        _publish(sid, cid, r0, acc, sum_out)

    return pl.kernel(body, out_type=out_type, mesh=_mesh(), scratch_types=scratch)


@functools.lru_cache(maxsize=None)
def _make_cnt():
    out_type = [jax.ShapeDtypeStruct((_NC, N_NODES, _D), jnp.float32)]
    scratch = [
        pltpu.VMEM_SHARED((N_NODES, _D), jnp.float32),
        pltpu.VMEM((_ZROWS, _D), jnp.float32),
        pltpu.VMEM((_CH, _D), jnp.float32),
        pltpu.VMEM((_CH,), jnp.int32),
    ]

    def body(dst_hbm, cnt_out, cacc, zbuf, ones, didx):
        cid = lax.axis_index("c")
        sid = lax.axis_index("s")
        wid = sid * _NC + cid
        r0 = sid * _RPT
        _zero_acc(sid, r0, zbuf, cacc)
        _fill_f32(ones, _CH, _D, 1.0)
        plsc.subcore_barrier()

        base = wid * _EPW

        def eloop(i, _):
            off = base + i * _CH
            pltpu.sync_copy(dst_hbm.at[pl.ds(off, _CH)], didx)
            pltpu.sync_copy(ones, cacc.at[didx], add=True)
            return 0

        lax.fori_loop(0, _NCHUNK, eloop, 0)
        plsc.subcore_barrier()
        _publish(sid, cid, r0, cacc, cnt_out)

    return pl.kernel(body, out_type=out_type, mesh=_mesh(), scratch_types=scratch)


_RB = 2000
_GRID = N_NODES // _RB


def _tc0_body(sum_ref, cnt_ref, x_ref, wl_ref, bl_ref, wr_ref, h_ref, st_ref):
    s = sum_ref[0] + sum_ref[1]
    c = cnt_ref[0, :, 0:1] + cnt_ref[1, :, 0:1]
    inv = 1.0 / jnp.maximum(c, 1.0)
    mean = s * inv
    h = (jnp.dot(mean, wl_ref[...], preferred_element_type=jnp.float32)
         + bl_ref[...]
         + jnp.dot(x_ref[...], wr_ref[...], preferred_element_type=jnp.float32))
    h_ref[...] = h

    @pl.when(pl.program_id(0) == 0)
    def _():
        st_ref[...] = jnp.zeros_like(st_ref)

    st_ref[0:1, :] += jnp.sum(h, axis=0, keepdims=True)
    st_ref[1:2, :] += jnp.sum(h * h, axis=0, keepdims=True)


_tc0 = pl.pallas_call(
    _tc0_body,
    grid=(_GRID,),
    in_specs=[
        pl.BlockSpec((_NC, _RB, _D), lambda i: (0, i, 0)),
        pl.BlockSpec((_NC, _RB, _D), lambda i: (0, i, 0)),
        pl.BlockSpec((_RB, D_IN), lambda i: (i, 0)),
        pl.BlockSpec((D_IN, D_HID), lambda i: (0, 0)),
        pl.BlockSpec((1, D_HID), lambda i: (0, 0)),
        pl.BlockSpec((D_IN, D_HID), lambda i: (0, 0)),
    ],
    out_specs=[
        pl.BlockSpec((_RB, D_HID), lambda i: (i, 0)),
        pl.BlockSpec((2, D_HID), lambda i: (0, 0)),
    ],
    out_shape=[
        jax.ShapeDtypeStruct((N_NODES, D_HID), jnp.float32),
        jax.ShapeDtypeStruct((2, D_HID), jnp.float32),
    ],
)


def _bn_body(h_ref, st_ref, g_ref, b_ref, o_ref):
    mu = st_ref[0:1, :] * (1.0 / N_NODES)
    var = st_ref[1:2, :] * (1.0 / N_NODES) - mu * mu
    o_ref[...] = jnp.maximum(
        (h_ref[...] - mu) * lax.rsqrt(var + EPS) * g_ref[...] + b_ref[...], 0.0)


_bn = pl.pallas_call(
    _bn_body,
    grid=(_GRID,),
    in_specs=[
        pl.BlockSpec((_RB, D_HID), lambda i: (i, 0)),
        pl.BlockSpec((2, D_HID), lambda i: (0, 0)),
        pl.BlockSpec((1, D_HID), lambda i: (0, 0)),
        pl.BlockSpec((1, D_HID), lambda i: (0, 0)),
    ],
    out_specs=pl.BlockSpec((_RB, D_HID), lambda i: (i, 0)),
    out_shape=jax.ShapeDtypeStruct((N_NODES, D_HID), jnp.float32),
)


def _tc1_body(sum_ref, cnt_ref, h_ref, wl_ref, bl_ref, wr_ref, o_ref):
    s = sum_ref[0] + sum_ref[1]
    c = cnt_ref[0, :, 0:1] + cnt_ref[1, :, 0:1]
    inv = 1.0 / jnp.maximum(c, 1.0)
    mean = s * inv
    o = (jnp.dot(mean, wl_ref[...], preferred_element_type=jnp.float32)
         + bl_ref[...]
         + jnp.dot(h_ref[...], wr_ref[...], preferred_element_type=jnp.float32))
    m = jnp.max(o, axis=-1, keepdims=True)
    z = o - m
    lse = jnp.log(jnp.sum(jnp.exp(z), axis=-1, keepdims=True))
    o_ref[...] = z - lse


_tc1 = pl.pallas_call(
    _tc1_body,
    grid=(_GRID,),
    in_specs=[
        pl.BlockSpec((_NC, _RB, _D), lambda i: (0, i, 0)),
        pl.BlockSpec((_NC, _RB, _D), lambda i: (0, i, 0)),
        pl.BlockSpec((_RB, D_HID), lambda i: (i, 0)),
        pl.BlockSpec((D_HID, D_OUT), lambda i: (0, 0)),
        pl.BlockSpec((1, D_OUT), lambda i: (0, 0)),
        pl.BlockSpec((D_HID, D_OUT), lambda i: (0, 0)),
    ],
    out_specs=pl.BlockSpec((_RB, D_OUT), lambda i: (i, 0)),
    out_shape=jax.ShapeDtypeStruct((N_NODES, D_OUT), jnp.float32),
)


def kernel(x, edge_index, Wl0, bl0, Wr0, gamma0, beta0, Wl1, bl1, Wr1):
    src = edge_index[0]
    dst = edge_index[1]
    (cntp,) = _make_cnt()(dst)
    (sum0p,) = _make_seg_sum()(x, src, dst)
    h_pre, stats = _tc0(sum0p, cntp, x, Wl0.T, bl0.reshape(1, -1), Wr0.T)
    h = _bn(h_pre, stats, gamma0.reshape(1, -1), beta0.reshape(1, -1))
    (sum1p,) = _make_seg_sum()(h, src, dst)
    return _tc1(sum1p, cntp, h, Wl1.T, bl1.reshape(1, -1), Wr1.T)


# speedup vs baseline: 10.2118x; 2.1453x over previous
import functools

import jax
import jax.numpy as jnp
from jax import lax
from jax.experimental import pallas as pl
from jax.experimental.pallas import tpu as pltpu
from jax.experimental.pallas import tpu_sc as plsc

N_NODES = 10000
N_EDGES = 320000
D_IN = 128
D_HID = 128
D_OUT = 64
EPS = 1e-5

_NC = 2
_NS = 16
_NW = _NC * _NS
_EPW = N_EDGES // _NW
_CH = 80
_NCHUNK = _EPW // _CH
_RPT = 624
_TAIL0 = _NS * _RPT
_ZROWS = 16
_D = 128


def _fill_f32(ref, rows, width, val):
    vec = jnp.full((16,), val, jnp.float32)

    def body(i, _):
        for k in range(width // 16):
            ref[i, pl.ds(k * 16, 16)] = vec
        return 0

    lax.fori_loop(0, rows, body, 0)


def _mesh():
    return plsc.VectorSubcoreMesh(core_axis_name="c", subcore_axis_name="s",
                                  num_cores=_NC, num_subcores=_NS)


def _zero_acc(sid, r0, zbuf, acc):
    _fill_f32(zbuf, _ZROWS, _D, 0.0)

    def zloop(j, _):
        pltpu.sync_copy(zbuf, acc.at[pl.ds(r0 + j * _ZROWS, _ZROWS)])
        return 0

    lax.fori_loop(0, _RPT // _ZROWS, zloop, 0)

    @pl.when(sid == _NS - 1)
    def _():
        pltpu.sync_copy(zbuf, acc.at[pl.ds(_TAIL0, _ZROWS)])


def _publish(sid, cid, r0, acc, out):
    pltpu.sync_copy(acc.at[pl.ds(r0, _RPT)], out.at[cid, pl.ds(r0, _RPT)])

    @pl.when(sid == _NS - 1)
    def _():
        pltpu.sync_copy(acc.at[pl.ds(_TAIL0, _ZROWS)],
                        out.at[cid, pl.ds(_TAIL0, _ZROWS)])


def _unpack_lo(pk, i, dst_ref, buf=None):
    for k in range(_CH // 16):
        v = pk[pl.ds(i * _CH + k * 16, 16)]
        lo = lax.bitwise_and(v, jnp.int32(0xFFFF))
        if buf is None:
            dst_ref[pl.ds(k * 16, 16)] = lo
        else:
            dst_ref[buf, pl.ds(k * 16, 16)] = lo


def _unpack_hi(pk, i, dst_ref):
    for k in range(_CH // 16):
        v = pk[pl.ds(i * _CH + k * 16, 16)]
        dst_ref[pl.ds(k * 16, 16)] = lax.shift_right_logical(v, 16)


@functools.lru_cache(maxsize=None)
def _make_seg_sum():
    out_type = [jax.ShapeDtypeStruct((_NC, N_NODES, _D), jnp.float32)]
    scratch = [
        pltpu.VMEM_SHARED((N_NODES, _D), jnp.float32),
        pltpu.VMEM((_ZROWS, _D), jnp.float32),
        pltpu.VMEM((_EPW,), jnp.int32),
        pltpu.VMEM((2, _CH), jnp.int32),
        pltpu.VMEM((_CH,), jnp.int32),
        pltpu.VMEM((2, _CH, _D), jnp.float32),
        pltpu.SemaphoreType.DMA,
    ]

    def body(x_hbm, pk_hbm, sum_out, acc, zbuf, pk, sidx, didx, rows, semg):
        cid = lax.axis_index("c")
        sid = lax.axis_index("s")
        wid = sid * _NC + cid
        r0 = sid * _RPT
        pltpu.sync_copy(pk_hbm.at[wid, 0], pk)
        _zero_acc(sid, r0, zbuf, acc)
        plsc.subcore_barrier()

        _unpack_lo(pk, 0, sidx, 0)
        pltpu.async_copy(x_hbm.at[sidx.at[0]], rows.at[0], semg)
        _unpack_lo(pk, 1, sidx, 1)
        pltpu.async_copy(x_hbm.at[sidx.at[1]], rows.at[1], semg)

        def eloop(i, _):
            buf = lax.rem(i, 2)
            pltpu.make_async_copy(x_hbm.at[sidx.at[buf]], rows.at[buf],
                                  semg).wait()
            _unpack_hi(pk, i, didx)
            pltpu.sync_copy(rows.at[buf], acc.at[didx], add=True)

            @pl.when(i + 2 < _NCHUNK)
            def _():
                _unpack_lo(pk, i + 2, sidx, buf)
                pltpu.async_copy(x_hbm.at[sidx.at[buf]], rows.at[buf], semg)

            return 0

        lax.fori_loop(0, _NCHUNK, eloop, 0)
        plsc.subcore_barrier()
        _publish(sid, cid, r0, acc, sum_out)

    return pl.kernel(body, out_type=out_type, mesh=_mesh(), scratch_types=scratch)


@functools.lru_cache(maxsize=None)
def _make_cnt():
    out_type = [jax.ShapeDtypeStruct((_NC, N_NODES, _D), jnp.float32)]
    scratch = [
        pltpu.VMEM_SHARED((N_NODES, _D), jnp.float32),
        pltpu.VMEM((_ZROWS, _D), jnp.float32),
        pltpu.VMEM((_CH, _D), jnp.float32),
        pltpu.VMEM((_EPW,), jnp.int32),
        pltpu.VMEM((_CH,), jnp.int32),
        pltpu.VMEM((_CH,), jnp.int32),
        pltpu.SemaphoreType.DMA,
    ]

    def body(pk_hbm, cnt_out, cacc, zbuf, ones, pk, didxa, didxb, sems):
        cid = lax.axis_index("c")
        sid = lax.axis_index("s")
        wid = sid * _NC + cid
        r0 = sid * _RPT
        pltpu.sync_copy(pk_hbm.at[wid, 0], pk)
        _zero_acc(sid, r0, zbuf, cacc)
        _fill_f32(ones, _CH, _D, 1.0)
        plsc.subcore_barrier()

        def eloop(i, _):
            @pl.when(lax.rem(i, 2) == 0)
            def _():
                _unpack_hi(pk, i, didxa)
                pltpu.async_copy(ones, cacc.at[didxa], sems, add=True)

            @pl.when(lax.rem(i, 2) == 1)
            def _():
                _unpack_hi(pk, i, didxb)
                pltpu.async_copy(ones, cacc.at[didxb], sems, add=True)

            @pl.when(i >= 1)
            def _():
                pltpu.make_async_copy(ones, cacc.at[didxa], sems).wait()

            return 0

        lax.fori_loop(0, _NCHUNK, eloop, 0)
        pltpu.make_async_copy(ones, cacc.at[didxa], sems).wait()
        plsc.subcore_barrier()
        _publish(sid, cid, r0, cacc, cnt_out)

    return pl.kernel(body, out_type=out_type, mesh=_mesh(), scratch_types=scratch)


_RB = 2000
_GRID = N_NODES // _RB


def _tc0_body(sum_ref, cnt_ref, x_ref, wl_ref, bl_ref, wr_ref, h_ref, st_ref):
    s = sum_ref[0] + sum_ref[1]
    c = cnt_ref[0, :, 0:1] + cnt_ref[1, :, 0:1]
    inv = 1.0 / jnp.maximum(c, 1.0)
    mean = s * inv
    h = (jnp.dot(mean, wl_ref[...], preferred_element_type=jnp.float32)
         + bl_ref[...]
         + jnp.dot(x_ref[...], wr_ref[...], preferred_element_type=jnp.float32))
    h_ref[...] = h

    @pl.when(pl.program_id(0) == 0)
    def _():
        st_ref[...] = jnp.zeros_like(st_ref)

    st_ref[0:1, :] += jnp.sum(h, axis=0, keepdims=True)
    st_ref[1:2, :] += jnp.sum(h * h, axis=0, keepdims=True)


_tc0 = pl.pallas_call(
    _tc0_body,
    grid=(_GRID,),
    in_specs=[
        pl.BlockSpec((_NC, _RB, _D), lambda i: (0, i, 0)),
        pl.BlockSpec((_NC, _RB, _D), lambda i: (0, i, 0)),
        pl.BlockSpec((_RB, D_IN), lambda i: (i, 0)),
        pl.BlockSpec((D_IN, D_HID), lambda i: (0, 0)),
        pl.BlockSpec((1, D_HID), lambda i: (0, 0)),
        pl.BlockSpec((D_IN, D_HID), lambda i: (0, 0)),
    ],
    out_specs=[
        pl.BlockSpec((_RB, D_HID), lambda i: (i, 0)),
        pl.BlockSpec((2, D_HID), lambda i: (0, 0)),
    ],
    out_shape=[
        jax.ShapeDtypeStruct((N_NODES, D_HID), jnp.float32),
        jax.ShapeDtypeStruct((2, D_HID), jnp.float32),
    ],
)


def _bn_body(h_ref, st_ref, g_ref, b_ref, o_ref):
    mu = st_ref[0:1, :] * (1.0 / N_NODES)
    var = st_ref[1:2, :] * (1.0 / N_NODES) - mu * mu
    o_ref[...] = jnp.maximum(
        (h_ref[...] - mu) * lax.rsqrt(var + EPS) * g_ref[...] + b_ref[...], 0.0)


_bn = pl.pallas_call(
    _bn_body,
    grid=(_GRID,),
    in_specs=[
        pl.BlockSpec((_RB, D_HID), lambda i: (i, 0)),
        pl.BlockSpec((2, D_HID), lambda i: (0, 0)),
        pl.BlockSpec((1, D_HID), lambda i: (0, 0)),
        pl.BlockSpec((1, D_HID), lambda i: (0, 0)),
    ],
    out_specs=pl.BlockSpec((_RB, D_HID), lambda i: (i, 0)),
    out_shape=jax.ShapeDtypeStruct((N_NODES, D_HID), jnp.float32),
)


def _tc1_body(sum_ref, cnt_ref, h_ref, wl_ref, bl_ref, wr_ref, o_ref):
    s = sum_ref[0] + sum_ref[1]
    c = cnt_ref[0, :, 0:1] + cnt_ref[1, :, 0:1]
    inv = 1.0 / jnp.maximum(c, 1.0)
    mean = s * inv
    o = (jnp.dot(mean, wl_ref[...], preferred_element_type=jnp.float32)
         + bl_ref[...]
         + jnp.dot(h_ref[...], wr_ref[...], preferred_element_type=jnp.float32))
    m = jnp.max(o, axis=-1, keepdims=True)
    z = o - m
    lse = jnp.log(jnp.sum(jnp.exp(z), axis=-1, keepdims=True))
    o_ref[...] = z - lse


_tc1 = pl.pallas_call(
    _tc1_body,
    grid=(_GRID,),
    in_specs=[
        pl.BlockSpec((_NC, _RB, _D), lambda i: (0, i, 0)),
        pl.BlockSpec((_NC, _RB, _D), lambda i: (0, i, 0)),
        pl.BlockSpec((_RB, D_HID), lambda i: (i, 0)),
        pl.BlockSpec((D_HID, D_OUT), lambda i: (0, 0)),
        pl.BlockSpec((1, D_OUT), lambda i: (0, 0)),
        pl.BlockSpec((D_HID, D_OUT), lambda i: (0, 0)),
    ],
    out_specs=pl.BlockSpec((_RB, D_OUT), lambda i: (i, 0)),
    out_shape=jax.ShapeDtypeStruct((N_NODES, D_OUT), jnp.float32),
)


def kernel(x, edge_index, Wl0, bl0, Wr0, gamma0, beta0, Wl1, bl1, Wr1):
    packed = (edge_index[0] | (edge_index[1] << 16)).reshape(_NW, 1, _EPW)
    (cntp,) = _make_cnt()(packed)
    (sum0p,) = _make_seg_sum()(x, packed)
    h_pre, stats = _tc0(sum0p, cntp, x, Wl0.T, bl0.reshape(1, -1), Wr0.T)
    h = _bn(h_pre, stats, gamma0.reshape(1, -1), beta0.reshape(1, -1))
    (sum1p,) = _make_seg_sum()(h, packed)
    return _tc1(sum1p, cntp, h, Wl1.T, bl1.reshape(1, -1), Wr1.T)


# speedup vs baseline: 11.5061x; 1.1267x over previous
import functools

import jax
import jax.numpy as jnp
from jax import lax
from jax.experimental import pallas as pl
from jax.experimental.pallas import tpu as pltpu
from jax.experimental.pallas import tpu_sc as plsc

N_NODES = 10000
N_EDGES = 320000
D_IN = 128
D_HID = 128
D_OUT = 64
EPS = 1e-5

_NC = 2
_NS = 16
_NW = _NC * _NS
_EPW = N_EDGES // _NW
_CH = 80
_NCHUNK = _EPW // _CH
_RPT = 624
_TAIL0 = _NS * _RPT
_ZROWS = 16
_D = 128


def _fill_f32(ref, rows, width, val):
    vec = jnp.full((16,), val, jnp.float32)

    def body(i, _):
        for k in range(width // 16):
            ref[i, pl.ds(k * 16, 16)] = vec
        return 0

    lax.fori_loop(0, rows, body, 0)


def _mesh():
    return plsc.VectorSubcoreMesh(core_axis_name="c", subcore_axis_name="s",
                                  num_cores=_NC, num_subcores=_NS)


def _zero_acc(sid, r0, zbuf, acc):
    _fill_f32(zbuf, _ZROWS, _D, 0.0)

    def zloop(j, _):
        pltpu.sync_copy(zbuf, acc.at[pl.ds(r0 + j * _ZROWS, _ZROWS)])
        return 0

    lax.fori_loop(0, _RPT // _ZROWS, zloop, 0)

    @pl.when(sid == _NS - 1)
    def _():
        pltpu.sync_copy(zbuf, acc.at[pl.ds(_TAIL0, _ZROWS)])


def _publish(sid, cid, r0, acc, out):
    pltpu.sync_copy(acc.at[pl.ds(r0, _RPT)], out.at[cid, pl.ds(r0, _RPT)])

    @pl.when(sid == _NS - 1)
    def _():
        pltpu.sync_copy(acc.at[pl.ds(_TAIL0, _ZROWS)],
                        out.at[cid, pl.ds(_TAIL0, _ZROWS)])


def _unpack_lo(pk, i, dst_ref, buf=None):
    for k in range(_CH // 16):
        v = pk[pl.ds(i * _CH + k * 16, 16)]
        lo = lax.bitwise_and(v, jnp.int32(0xFFFF))
        if buf is None:
            dst_ref[pl.ds(k * 16, 16)] = lo
        else:
            dst_ref[buf, pl.ds(k * 16, 16)] = lo


def _unpack_hi(pk, i, dst_ref):
    for k in range(_CH // 16):
        v = pk[pl.ds(i * _CH + k * 16, 16)]
        dst_ref[pl.ds(k * 16, 16)] = lax.shift_right_logical(v, 16)


@functools.lru_cache(maxsize=None)
def _make_seg_sum():
    out_type = [jax.ShapeDtypeStruct((_NC, N_NODES, _D), jnp.float32)]
    scratch = [
        pltpu.VMEM_SHARED((N_NODES, _D), jnp.float32),
        pltpu.VMEM((_ZROWS, _D), jnp.float32),
        pltpu.VMEM((_EPW,), jnp.int32),
        pltpu.VMEM((2, _CH), jnp.int32),
        pltpu.VMEM((_CH,), jnp.int32),
        pltpu.VMEM((_CH,), jnp.int32),
        pltpu.VMEM((3, _CH, _D), jnp.float32),
        pltpu.SemaphoreType.DMA,
        pltpu.SemaphoreType.DMA,
    ]

    def body(x_hbm, pk_hbm, sum_out, acc, zbuf, pk, sidx, didxa, didxb,
             rows, semg, sems):
        cid = lax.axis_index("c")
        sid = lax.axis_index("s")
        wid = sid * _NC + cid
        r0 = sid * _RPT
        pltpu.sync_copy(pk_hbm.at[wid, 0], pk)
        _zero_acc(sid, r0, zbuf, acc)
        plsc.subcore_barrier()

        _unpack_lo(pk, 0, sidx, 0)
        pltpu.async_copy(x_hbm.at[sidx.at[0]], rows.at[0], semg)
        _unpack_lo(pk, 1, sidx, 1)
        pltpu.async_copy(x_hbm.at[sidx.at[1]], rows.at[1], semg)

        def eloop(i, _):
            gbuf = lax.rem(i, 3)
            pltpu.make_async_copy(x_hbm.at[sidx.at[0]], rows.at[gbuf],
                                  semg).wait()

            @pl.when(lax.rem(i, 2) == 0)
            def _():
                _unpack_hi(pk, i, didxa)
                pltpu.async_copy(rows.at[gbuf], acc.at[didxa], sems, add=True)

            @pl.when(lax.rem(i, 2) == 1)
            def _():
                _unpack_hi(pk, i, didxb)
                pltpu.async_copy(rows.at[gbuf], acc.at[didxb], sems, add=True)

            @pl.when(i >= 1)
            def _():
                pltpu.make_async_copy(rows.at[0], acc.at[didxa], sems).wait()

            @pl.when(i + 2 < _NCHUNK)
            def _():
                sbuf = lax.rem(i, 2)
                _unpack_lo(pk, i + 2, sidx, sbuf)
                pltpu.async_copy(x_hbm.at[sidx.at[sbuf]],
                                 rows.at[lax.rem(i + 2, 3)], semg)

            return 0

        lax.fori_loop(0, _NCHUNK, eloop, 0)
        pltpu.make_async_copy(rows.at[0], acc.at[didxa], sems).wait()
        plsc.subcore_barrier()
        _publish(sid, cid, r0, acc, sum_out)

    return pl.kernel(body, out_type=out_type, mesh=_mesh(), scratch_types=scratch)


@functools.lru_cache(maxsize=None)
def _make_cnt():
    out_type = [jax.ShapeDtypeStruct((_NC, N_NODES, _D), jnp.float32)]
    scratch = [
        pltpu.VMEM_SHARED((N_NODES, _D), jnp.float32),
        pltpu.VMEM((_ZROWS, _D), jnp.float32),
        pltpu.VMEM((_CH, _D), jnp.float32),
        pltpu.VMEM((_EPW,), jnp.int32),
        pltpu.VMEM((_CH,), jnp.int32),
        pltpu.VMEM((_CH,), jnp.int32),
        pltpu.VMEM((_CH,), jnp.int32),
        pltpu.SemaphoreType.DMA,
    ]

    def body(pk_hbm, cnt_out, cacc, zbuf, ones, pk, didxa, didxb, didxc, sems):
        cid = lax.axis_index("c")
        sid = lax.axis_index("s")
        wid = sid * _NC + cid
        r0 = sid * _RPT
        pltpu.sync_copy(pk_hbm.at[wid, 0], pk)
        _zero_acc(sid, r0, zbuf, cacc)
        _fill_f32(ones, _CH, _D, 1.0)
        plsc.subcore_barrier()

        def eloop(i, _):
            @pl.when(lax.rem(i, 3) == 0)
            def _():
                _unpack_hi(pk, i, didxa)
                pltpu.async_copy(ones, cacc.at[didxa], sems, add=True)

            @pl.when(lax.rem(i, 3) == 1)
            def _():
                _unpack_hi(pk, i, didxb)
                pltpu.async_copy(ones, cacc.at[didxb], sems, add=True)

            @pl.when(lax.rem(i, 3) == 2)
            def _():
                _unpack_hi(pk, i, didxc)
                pltpu.async_copy(ones, cacc.at[didxc], sems, add=True)

            @pl.when(i >= 2)
            def _():
                pltpu.make_async_copy(ones, cacc.at[didxa], sems).wait()

            return 0

        lax.fori_loop(0, _NCHUNK, eloop, 0)
        pltpu.make_async_copy(ones, cacc.at[didxa], sems).wait()
        pltpu.make_async_copy(ones, cacc.at[didxa], sems).wait()
        plsc.subcore_barrier()
        _publish(sid, cid, r0, cacc, cnt_out)

    return pl.kernel(body, out_type=out_type, mesh=_mesh(), scratch_types=scratch)


_RB = 2000
_GRID = N_NODES // _RB


def _tc0_body(sum_ref, cnt_ref, x_ref, wl_ref, bl_ref, wr_ref, g_ref, be_ref,
              h_ref, inv_ref, hpre_ref, st_ref):
    p = pl.program_id(0)
    i = pl.program_id(1)

    @pl.when(p == 0)
    def _():
        s = sum_ref[0] + sum_ref[1]
        c = cnt_ref[0, :, 0:1] + cnt_ref[1, :, 0:1]
        inv = 1.0 / jnp.maximum(c, 1.0)
        inv_ref[...] = inv
        mean = s * inv
        h = (jnp.dot(mean, wl_ref[...], preferred_element_type=jnp.float32)
             + bl_ref[...]
             + jnp.dot(x_ref[...], wr_ref[...],
                       preferred_element_type=jnp.float32))
        hpre_ref[pl.ds(i * _RB, _RB), :] = h

        @pl.when(i == 0)
        def _():
            st_ref[...] = jnp.zeros_like(st_ref)

        st_ref[0:1, :] += jnp.sum(h, axis=0, keepdims=True)
        st_ref[1:2, :] += jnp.sum(h * h, axis=0, keepdims=True)

    @pl.when(p == 1)
    def _():
        mu = st_ref[0:1, :] * (1.0 / N_NODES)
        var = st_ref[1:2, :] * (1.0 / N_NODES) - mu * mu
        hp = hpre_ref[pl.ds(i * _RB, _RB), :]
        h_ref[...] = jnp.maximum(
            (hp - mu) * lax.rsqrt(var + EPS) * g_ref[...] + be_ref[...], 0.0)


_tc0 = pl.pallas_call(
    _tc0_body,
    grid=(2, _GRID),
    in_specs=[
        pl.BlockSpec((_NC, _RB, _D), lambda p, i: (0, jnp.where(p == 0, i, _GRID - 1), 0)),
        pl.BlockSpec((_NC, _RB, _D), lambda p, i: (0, jnp.where(p == 0, i, _GRID - 1), 0)),
        pl.BlockSpec((_RB, D_IN), lambda p, i: (jnp.where(p == 0, i, _GRID - 1), 0)),
        pl.BlockSpec((D_IN, D_HID), lambda p, i: (0, 0)),
        pl.BlockSpec((1, D_HID), lambda p, i: (0, 0)),
        pl.BlockSpec((D_IN, D_HID), lambda p, i: (0, 0)),
        pl.BlockSpec((1, D_HID), lambda p, i: (0, 0)),
        pl.BlockSpec((1, D_HID), lambda p, i: (0, 0)),
    ],
    out_specs=[
        pl.BlockSpec((_RB, D_HID), lambda p, i: (jnp.where(p == 0, 0, i), 0)),
        pl.BlockSpec((_RB, 1), lambda p, i: (jnp.where(p == 0, i, _GRID - 1), 0)),
    ],
    out_shape=[
        jax.ShapeDtypeStruct((N_NODES, D_HID), jnp.float32),
        jax.ShapeDtypeStruct((N_NODES, 1), jnp.float32),
    ],
    scratch_shapes=[
        pltpu.VMEM((N_NODES, D_HID), jnp.float32),
        pltpu.VMEM((2, D_HID), jnp.float32),
    ],
)


def _tc1_body(sum_ref, inv_ref, h_ref, wl_ref, bl_ref, wr_ref, o_ref):
    s = sum_ref[0] + sum_ref[1]
    mean = s * inv_ref[...]
    o = (jnp.dot(mean, wl_ref[...], preferred_element_type=jnp.float32)
         + bl_ref[...]
         + jnp.dot(h_ref[...], wr_ref[...], preferred_element_type=jnp.float32))
    m = jnp.max(o, axis=-1, keepdims=True)
    z = o - m
    lse = jnp.log(jnp.sum(jnp.exp(z), axis=-1, keepdims=True))
    o_ref[...] = z - lse


_tc1 = pl.pallas_call(
    _tc1_body,
    grid=(_GRID,),
    in_specs=[
        pl.BlockSpec((_NC, _RB, _D), lambda i: (0, i, 0)),
        pl.BlockSpec((_RB, 1), lambda i: (i, 0)),
        pl.BlockSpec((_RB, D_HID), lambda i: (i, 0)),
        pl.BlockSpec((D_HID, D_OUT), lambda i: (0, 0)),
        pl.BlockSpec((1, D_OUT), lambda i: (0, 0)),
        pl.BlockSpec((D_HID, D_OUT), lambda i: (0, 0)),
    ],
    out_specs=pl.BlockSpec((_RB, D_OUT), lambda i: (i, 0)),
    out_shape=jax.ShapeDtypeStruct((N_NODES, D_OUT), jnp.float32),
)


def kernel(x, edge_index, Wl0, bl0, Wr0, gamma0, beta0, Wl1, bl1, Wr1):
    packed = (edge_index[0] | (edge_index[1] << 16)).reshape(_NW, 1, _EPW)
    (cntp,) = _make_cnt()(packed)
    (sum0p,) = _make_seg_sum()(x, packed)
    h, inv = _tc0(sum0p, cntp, x, Wl0.T, bl0.reshape(1, -1), Wr0.T,
                  gamma0.reshape(1, -1), beta0.reshape(1, -1))
    (sum1p,) = _make_seg_sum()(h, packed)
    return _tc1(sum1p, inv, h, Wl1.T, bl1.reshape(1, -1), Wr1.T)


# speedup vs baseline: 11.6787x; 1.0150x over previous
import functools

import jax
import jax.numpy as jnp
from jax import lax
from jax.experimental import pallas as pl
from jax.experimental.pallas import tpu as pltpu
from jax.experimental.pallas import tpu_sc as plsc

N_NODES = 10000
N_EDGES = 320000
D_IN = 128
D_HID = 128
D_OUT = 64
EPS = 1e-5

_NC = 2
_NS = 16
_NW = _NC * _NS
_EPW = N_EDGES // _NW
_CH = 80
_NCHUNK = _EPW // _CH
_RPT = 624
_TAIL0 = _NS * _RPT
_ZROWS = 16
_D = 128


def _fill_f32(ref, rows, width, val):
    vec = jnp.full((16,), val, jnp.float32)

    def body(i, _):
        for k in range(width // 16):
            ref[i, pl.ds(k * 16, 16)] = vec
        return 0

    lax.fori_loop(0, rows, body, 0)


def _mesh():
    return plsc.VectorSubcoreMesh(core_axis_name="c", subcore_axis_name="s",
                                  num_cores=_NC, num_subcores=_NS)


def _zero_acc(sid, r0, zbuf, acc):
    _fill_f32(zbuf, _ZROWS, _D, 0.0)

    def zloop(j, _):
        pltpu.sync_copy(zbuf, acc.at[pl.ds(r0 + j * _ZROWS, _ZROWS)])
        return 0

    lax.fori_loop(0, _RPT // _ZROWS, zloop, 0)

    @pl.when(sid == _NS - 1)
    def _():
        pltpu.sync_copy(zbuf, acc.at[pl.ds(_TAIL0, _ZROWS)])


def _publish(sid, cid, r0, acc, out):
    pltpu.sync_copy(acc.at[pl.ds(r0, _RPT)], out.at[cid, pl.ds(r0, _RPT)])

    @pl.when(sid == _NS - 1)
    def _():
        pltpu.sync_copy(acc.at[pl.ds(_TAIL0, _ZROWS)],
                        out.at[cid, pl.ds(_TAIL0, _ZROWS)])


def _unpack_lo(pk, i, dst_ref, buf=None):
    for k in range(_CH // 16):
        v = pk[pl.ds(i * _CH + k * 16, 16)]
        lo = lax.bitwise_and(v, jnp.int32(0xFFFF))
        if buf is None:
            dst_ref[pl.ds(k * 16, 16)] = lo
        else:
            dst_ref[buf, pl.ds(k * 16, 16)] = lo


def _unpack_hi(pk, i, dst_ref):
    for k in range(_CH // 16):
        v = pk[pl.ds(i * _CH + k * 16, 16)]
        dst_ref[pl.ds(k * 16, 16)] = lax.shift_right_logical(v, 16)


def _seg_phase(x_hbm, acc, pk, sidx, didxa, didxb, rows, semg, sems):
    _unpack_lo(pk, 0, sidx, 0)
    pltpu.async_copy(x_hbm.at[sidx.at[0]], rows.at[0], semg)
    _unpack_lo(pk, 1, sidx, 1)
    pltpu.async_copy(x_hbm.at[sidx.at[1]], rows.at[1], semg)

    def eloop(i, _):
        gbuf = lax.rem(i, 3)
        pltpu.make_async_copy(x_hbm.at[sidx.at[0]], rows.at[gbuf], semg).wait()

        @pl.when(lax.rem(i, 2) == 0)
        def _():
            _unpack_hi(pk, i, didxa)
            pltpu.async_copy(rows.at[gbuf], acc.at[didxa], sems, add=True)

        @pl.when(lax.rem(i, 2) == 1)
        def _():
            _unpack_hi(pk, i, didxb)
            pltpu.async_copy(rows.at[gbuf], acc.at[didxb], sems, add=True)

        @pl.when(i >= 1)
        def _():
            pltpu.make_async_copy(rows.at[0], acc.at[didxa], sems).wait()

        @pl.when(i + 2 < _NCHUNK)
        def _():
            sbuf = lax.rem(i, 2)
            _unpack_lo(pk, i + 2, sidx, sbuf)
            pltpu.async_copy(x_hbm.at[sidx.at[sbuf]],
                             rows.at[lax.rem(i + 2, 3)], semg)

        return 0

    lax.fori_loop(0, _NCHUNK, eloop, 0)
    pltpu.make_async_copy(rows.at[0], acc.at[didxa], sems).wait()


def _cnt_phase(acc, pk, ones_src, didxa, didxb, didxc, sems):
    def eloop(i, _):
        @pl.when(lax.rem(i, 3) == 0)
        def _():
            _unpack_hi(pk, i, didxa)
            pltpu.async_copy(ones_src, acc.at[didxa], sems, add=True)

        @pl.when(lax.rem(i, 3) == 1)
        def _():
            _unpack_hi(pk, i, didxb)
            pltpu.async_copy(ones_src, acc.at[didxb], sems, add=True)

        @pl.when(lax.rem(i, 3) == 2)
        def _():
            _unpack_hi(pk, i, didxc)
            pltpu.async_copy(ones_src, acc.at[didxc], sems, add=True)

        @pl.when(i >= 2)
        def _():
            pltpu.make_async_copy(ones_src, acc.at[didxa], sems).wait()

        return 0

    lax.fori_loop(0, _NCHUNK, eloop, 0)
    pltpu.make_async_copy(ones_src, acc.at[didxa], sems).wait()
    pltpu.make_async_copy(ones_src, acc.at[didxa], sems).wait()


_CW = _D


def _sc_scratch():
    return [
        pltpu.VMEM_SHARED((N_NODES, _D), jnp.float32),
        pltpu.VMEM((_ZROWS, _D), jnp.float32),
        pltpu.VMEM((_EPW,), jnp.int32),
        pltpu.VMEM((2, _CH), jnp.int32),
        pltpu.VMEM((_CH,), jnp.int32),
        pltpu.VMEM((_CH,), jnp.int32),
        pltpu.VMEM((_CH,), jnp.int32),
        pltpu.VMEM((3, _CH, _D), jnp.float32),
        pltpu.SemaphoreType.DMA,
        pltpu.SemaphoreType.DMA,
    ]


@functools.lru_cache(maxsize=None)
def _make_seg0_cnt():
    out_type = [jax.ShapeDtypeStruct((_NC, N_NODES, _D), jnp.float32),
                jax.ShapeDtypeStruct((_NC, N_NODES, _CW), jnp.float32)]

    def body(x_hbm, pk_hbm, sum_out, cnt_out, acc, zbuf, pk, sidx,
             didxa, didxb, didxc, rows, semg, sems):
        cid = lax.axis_index("c")
        sid = lax.axis_index("s")
        wid = sid * _NC + cid
        r0 = sid * _RPT
        pltpu.sync_copy(pk_hbm.at[wid, 0], pk)
        _zero_acc(sid, r0, zbuf, acc)
        _fill_f32(rows.at[2], _CH, _D, 1.0)
        plsc.subcore_barrier()
        _cnt_phase(acc, pk, rows.at[2], didxa, didxb, didxc, sems)
        plsc.subcore_barrier()
        _publish(sid, cid, r0, acc, cnt_out)
        _zero_acc(sid, r0, zbuf, acc)
        plsc.subcore_barrier()
        _seg_phase(x_hbm, acc, pk, sidx, didxa, didxb, rows, semg, sems)
        plsc.subcore_barrier()
        _publish(sid, cid, r0, acc, sum_out)

    return pl.kernel(body, out_type=out_type, mesh=_mesh(),
                     scratch_types=_sc_scratch())


@functools.lru_cache(maxsize=None)
def _make_seg_sum():
    out_type = [jax.ShapeDtypeStruct((_NC, N_NODES, _D), jnp.float32)]

    def body(x_hbm, pk_hbm, sum_out, acc, zbuf, pk, sidx,
             didxa, didxb, didxc, rows, semg, sems):
        cid = lax.axis_index("c")
        sid = lax.axis_index("s")
        wid = sid * _NC + cid
        r0 = sid * _RPT
        pltpu.sync_copy(pk_hbm.at[wid, 0], pk)
        _zero_acc(sid, r0, zbuf, acc)
        plsc.subcore_barrier()
        _seg_phase(x_hbm, acc, pk, sidx, didxa, didxb, rows, semg, sems)
        plsc.subcore_barrier()
        _publish(sid, cid, r0, acc, sum_out)

    return pl.kernel(body, out_type=out_type, mesh=_mesh(),
                     scratch_types=_sc_scratch())


_RB = 2000
_GRID = N_NODES // _RB


def _tc0_body(sum_ref, cnt_ref, x_ref, wl_ref, bl_ref, wr_ref, g_ref, be_ref,
              h_ref, inv_ref, hpre_ref, st_ref):
    p = pl.program_id(0)
    i = pl.program_id(1)

    @pl.when(p == 0)
    def _():
        s = sum_ref[0] + sum_ref[1]
        c = cnt_ref[0, :, 0:1] + cnt_ref[1, :, 0:1]
        inv = 1.0 / jnp.maximum(c, 1.0)
        inv_ref[...] = inv
        mean = s * inv
        h = (jnp.dot(mean, wl_ref[...], preferred_element_type=jnp.float32)
             + bl_ref[...]
             + jnp.dot(x_ref[...], wr_ref[...],
                       preferred_element_type=jnp.float32))
        hpre_ref[pl.ds(i * _RB, _RB), :] = h

        @pl.when(i == 0)
        def _():
            st_ref[...] = jnp.zeros_like(st_ref)

        st_ref[0:1, :] += jnp.sum(h, axis=0, keepdims=True)
        st_ref[1:2, :] += jnp.sum(h * h, axis=0, keepdims=True)

    @pl.when(p == 1)
    def _():
        mu = st_ref[0:1, :] * (1.0 / N_NODES)
        var = st_ref[1:2, :] * (1.0 / N_NODES) - mu * mu
        hp = hpre_ref[pl.ds(i * _RB, _RB), :]
        h_ref[...] = jnp.maximum(
            (hp - mu) * lax.rsqrt(var + EPS) * g_ref[...] + be_ref[...], 0.0)


_tc0 = pl.pallas_call(
    _tc0_body,
    grid=(2, _GRID),
    in_specs=[
        pl.BlockSpec((_NC, _RB, _D), lambda p, i: (0, jnp.where(p == 0, i, _GRID - 1), 0)),
        pl.BlockSpec((_NC, _RB, _CW), lambda p, i: (0, jnp.where(p == 0, i, _GRID - 1), 0)),
        pl.BlockSpec((_RB, D_IN), lambda p, i: (jnp.where(p == 0, i, _GRID - 1), 0)),
        pl.BlockSpec((D_IN, D_HID), lambda p, i: (0, 0)),
        pl.BlockSpec((1, D_HID), lambda p, i: (0, 0)),
        pl.BlockSpec((D_IN, D_HID), lambda p, i: (0, 0)),
        pl.BlockSpec((1, D_HID), lambda p, i: (0, 0)),
        pl.BlockSpec((1, D_HID), lambda p, i: (0, 0)),
    ],
    out_specs=[
        pl.BlockSpec((_RB, D_HID), lambda p, i: (jnp.where(p == 0, 0, i), 0)),
        pl.BlockSpec((_RB, 1), lambda p, i: (jnp.where(p == 0, i, _GRID - 1), 0)),
    ],
    out_shape=[
        jax.ShapeDtypeStruct((N_NODES, D_HID), jnp.float32),
        jax.ShapeDtypeStruct((N_NODES, 1), jnp.float32),
    ],
    scratch_shapes=[
        pltpu.VMEM((N_NODES, D_HID), jnp.float32),
        pltpu.VMEM((2, D_HID), jnp.float32),
    ],
)


def _tc1_body(sum_ref, inv_ref, h_ref, wl_ref, bl_ref, wr_ref, o_ref):
    s = sum_ref[0] + sum_ref[1]
    mean = s * inv_ref[...]
    o = (jnp.dot(mean, wl_ref[...], preferred_element_type=jnp.float32)
         + bl_ref[...]
         + jnp.dot(h_ref[...], wr_ref[...], preferred_element_type=jnp.float32))
    m = jnp.max(o, axis=-1, keepdims=True)
    z = o - m
    lse = jnp.log(jnp.sum(jnp.exp(z), axis=-1, keepdims=True))
    o_ref[...] = z - lse


_tc1 = pl.pallas_call(
    _tc1_body,
    grid=(_GRID,),
    in_specs=[
        pl.BlockSpec((_NC, _RB, _D), lambda i: (0, i, 0)),
        pl.BlockSpec((_RB, 1), lambda i: (i, 0)),
        pl.BlockSpec((_RB, D_HID), lambda i: (i, 0)),
        pl.BlockSpec((D_HID, D_OUT), lambda i: (0, 0)),
        pl.BlockSpec((1, D_OUT), lambda i: (0, 0)),
        pl.BlockSpec((D_HID, D_OUT), lambda i: (0, 0)),
    ],
    out_specs=pl.BlockSpec((_RB, D_OUT), lambda i: (i, 0)),
    out_shape=jax.ShapeDtypeStruct((N_NODES, D_OUT), jnp.float32),
)


def kernel(x, edge_index, Wl0, bl0, Wr0, gamma0, beta0, Wl1, bl1, Wr1):
    packed = (edge_index[0] | (edge_index[1] << 16)).reshape(_NW, 1, _EPW)
    sum0p, cntp = _make_seg0_cnt()(x, packed)
    h, inv = _tc0(sum0p, cntp, x, Wl0.T, bl0.reshape(1, -1), Wr0.T,
                  gamma0.reshape(1, -1), beta0.reshape(1, -1))
    (sum1p,) = _make_seg_sum()(h, packed)
    return _tc1(sum1p, inv, h, Wl1.T, bl1.reshape(1, -1), Wr1.T)
